# Initial kernel scaffold; baseline (speedup 1.0000x reference)
import functools

import jax
import jax.numpy as jnp
from jax import lax
from jax.experimental import pallas as pl
from jax.experimental.pallas import tpu as pltpu
from jax.experimental.pallas import tpu_sc as plsc

N_NODES = 10000
N_EDGES = 320000
D = 128

NC = 2
NS = 16
NW = NC * NS
K = 128
EDGES_PER_W = (2 * N_EDGES) // NW
CHUNKS = 158
PAD_W = CHUNKS * K - EDGES_PER_W
ACC_ROWS = 10016
TRASH_ROW = N_NODES
ROWS_PER_SUB = ACC_ROWS // NS


def _sc_segment_sum(table, gidx, didx, zeros_blk):
    mesh = plsc.VectorSubcoreMesh(core_axis_name="c", subcore_axis_name="s")

    @functools.partial(
        pl.kernel,
        out_type=jax.ShapeDtypeStruct((NC, ACC_ROWS, D), jnp.float32),
        mesh=mesh,
        scratch_types=[
            pltpu.VMEM((CHUNKS, K), jnp.int32),
            pltpu.VMEM((CHUNKS, K), jnp.int32),
            pltpu.VMEM((2, K, D), jnp.float32),
            pltpu.VMEM_SHARED((ACC_ROWS, D), jnp.float32),
            pltpu.SemaphoreType.DMA,
            pltpu.SemaphoreType.DMA,
        ],
    )
    def kern(t_hbm, g_hbm, d_hbm, z_hbm, out_hbm,
             g_v, d_v, rows_v, acc, semA, semB):
        c = lax.axis_index("c")
        s = lax.axis_index("s")
        wid = s * NC + c

        pltpu.sync_copy(z_hbm, acc.at[pl.ds(s * ROWS_PER_SUB, ROWS_PER_SUB)])
        pltpu.sync_copy(g_hbm.at[wid], g_v)
        pltpu.sync_copy(d_hbm.at[wid], d_v)

        pltpu.async_copy(t_hbm.at[g_v.at[0]], rows_v.at[0], semA)
        pltpu.async_copy(t_hbm.at[g_v.at[1]], rows_v.at[1], semB)

        plsc.subcore_barrier()

        @pl.loop(0, CHUNKS, step=2)
        def _(ci):
            pltpu.make_async_copy(
                t_hbm.at[g_v.at[ci]], rows_v.at[0], semA).wait()
            pltpu.sync_copy(rows_v.at[0], acc.at[d_v.at[ci]], add=True)

            @pl.when(ci + 2 < CHUNKS)
            def _():
                pltpu.async_copy(t_hbm.at[g_v.at[ci + 2]], rows_v.at[0], semA)

            pltpu.make_async_copy(
                t_hbm.at[g_v.at[ci + 1]], rows_v.at[1], semB).wait()
            pltpu.sync_copy(rows_v.at[1], acc.at[d_v.at[ci + 1]], add=True)

            @pl.when(ci + 3 < CHUNKS)
            def _():
                pltpu.async_copy(t_hbm.at[g_v.at[ci + 3]], rows_v.at[1], semB)

        plsc.subcore_barrier()
        pltpu.sync_copy(
            acc.at[pl.ds(s * ROWS_PER_SUB, ROWS_PER_SUB)],
            out_hbm.at[c, pl.ds(s * ROWS_PER_SUB, ROWS_PER_SUB)])

    return kern(table, gidx, didx, zeros_blk)


def _tc_mlp_bn(parts, node_feats, W1, b1, W2, b2, Wres, bres, gamma, beta):
    def body(parts_r, nf_r, W1_r, b1_r, W2_r, b2_r, Wres_r, bres_r,
             gamma_r, beta_r, out_r):
        agg = parts_r[0, :N_NODES, :] + parts_r[1, :N_NODES, :]
        h1 = jnp.maximum(
            jnp.dot(agg, W1_r[...], preferred_element_type=jnp.float32)
            + b1_r[...], 0.0)
        h = (jnp.dot(h1, W2_r[...], preferred_element_type=jnp.float32)
             + b2_r[...])
        res = (jnp.dot(nf_r[...], Wres_r[...],
                       preferred_element_type=jnp.float32) + bres_r[...])
        h = h + res
        mean = jnp.mean(h, axis=0, keepdims=True)
        var = jnp.mean((h - mean) ** 2, axis=0, keepdims=True)
        out_r[...] = ((h - mean) * lax.rsqrt(var + 1e-5) * gamma_r[...]
                      + beta_r[...])

    return pl.pallas_call(
        body,
        out_shape=jax.ShapeDtypeStruct((N_NODES, D), jnp.float32),
    )(parts, node_feats, W1, b1, W2, b2, Wres, bres, gamma, beta)


@jax.jit
def kernel(node_feats, edge_index, edge_feat_0, edge_feat_1,
           emb0, emb1, W1, b1, W2, b2, Wres, bres, gamma, beta):
    src = edge_index[0].astype(jnp.int32)
    dst = edge_index[1].astype(jnp.int32)
    eidx = (N_NODES + edge_feat_0.astype(jnp.int32) * 3
            + edge_feat_1.astype(jnp.int32))

    emb_pair = (emb0[:, None, :] + emb1[None, :, :]).reshape(18, D)
    table = jnp.concatenate([node_feats, emb_pair], axis=0)

    per = N_EDGES // NW
    g_pad = jnp.zeros((NW, PAD_W), jnp.int32)
    d_pad = jnp.full((NW, PAD_W), TRASH_ROW, jnp.int32)
    gidx = jnp.concatenate(
        [src.reshape(NW, per), eidx.reshape(NW, per), g_pad],
        axis=1).reshape(NW, CHUNKS, K)
    didx = jnp.concatenate(
        [dst.reshape(NW, per), dst.reshape(NW, per), d_pad],
        axis=1).reshape(NW, CHUNKS, K)

    zeros_blk = jnp.zeros((ROWS_PER_SUB, D), jnp.float32)
    parts = _sc_segment_sum(table, gidx, didx, zeros_blk)

    b1_2 = b1.reshape(1, 2 * D)
    b2_2 = b2.reshape(1, D)
    bres_2 = bres.reshape(1, D)
    gamma_2 = gamma.reshape(1, D)
    beta_2 = beta.reshape(1, D)
    return _tc_mlp_bn(parts, node_feats, W1, b1_2, W2, b2_2,
                      Wres, bres_2, gamma_2, beta_2)


# speedup vs baseline: 2.4927x; 2.4927x over previous
import functools

import jax
import jax.numpy as jnp
from jax import lax
from jax.experimental import pallas as pl
from jax.experimental.pallas import tpu as pltpu
from jax.experimental.pallas import tpu_sc as plsc

N_NODES = 10000
N_EDGES = 320000
D = 128

NC = 2
NS = 16
NW = NC * NS
K = 128
EDGES_PER_W = (2 * N_EDGES) // NW
IB = 16
GROUPS = 10
CHUNKS = IB * GROUPS
PAD_W = CHUNKS * K - EDGES_PER_W
ACC_ROWS = 10112
TRASH_ROW = N_NODES
ROWS_PER_SUB = ACC_ROWS // NS


def _sc_segment_sum(table, gidx, didx, zeros_blk):
    mesh = plsc.VectorSubcoreMesh(core_axis_name="c", subcore_axis_name="s")

    @functools.partial(
        pl.kernel,
        out_type=jax.ShapeDtypeStruct((NC, ACC_ROWS, D), jnp.float32),
        mesh=mesh,
        scratch_types=[
            pltpu.VMEM((IB, K), jnp.int32),
            pltpu.VMEM((IB, K), jnp.int32),
            pltpu.VMEM((2, K, D), jnp.float32),
            pltpu.VMEM_SHARED((ACC_ROWS, D), jnp.float32),
            pltpu.SemaphoreType.DMA,
            pltpu.SemaphoreType.DMA,
        ],
    )
    def kern(t_hbm, g_hbm, d_hbm, z_hbm, out_hbm,
             g_v, d_v, rows_v, acc, semA, semB):
        c = lax.axis_index("c")
        s = lax.axis_index("s")
        wid = s * NC + c

        pltpu.sync_copy(z_hbm, acc.at[pl.ds(s * ROWS_PER_SUB, ROWS_PER_SUB)])
        plsc.subcore_barrier()

        @pl.loop(0, GROUPS)
        def _(g):
            pltpu.sync_copy(g_hbm.at[wid, pl.ds(g * IB, IB)], g_v)
            pltpu.sync_copy(d_hbm.at[wid, pl.ds(g * IB, IB)], d_v)
            pltpu.async_copy(t_hbm.at[g_v.at[0]], rows_v.at[0], semA)
            pltpu.async_copy(t_hbm.at[g_v.at[1]], rows_v.at[1], semB)

            @pl.loop(0, IB, step=2)
            def _(ci):
                pltpu.make_async_copy(
                    t_hbm.at[g_v.at[ci]], rows_v.at[0], semA).wait()
                pltpu.sync_copy(rows_v.at[0], acc.at[d_v.at[ci]], add=True)

                @pl.when(ci + 2 < IB)
                def _():
                    pltpu.async_copy(
                        t_hbm.at[g_v.at[ci + 2]], rows_v.at[0], semA)

                pltpu.make_async_copy(
                    t_hbm.at[g_v.at[ci + 1]], rows_v.at[1], semB).wait()
                pltpu.sync_copy(rows_v.at[1], acc.at[d_v.at[ci + 1]], add=True)

                @pl.when(ci + 3 < IB)
                def _():
                    pltpu.async_copy(
                        t_hbm.at[g_v.at[ci + 3]], rows_v.at[1], semB)

        plsc.subcore_barrier()
        pltpu.sync_copy(
            acc.at[pl.ds(s * ROWS_PER_SUB, ROWS_PER_SUB)],
            out_hbm.at[c, pl.ds(s * ROWS_PER_SUB, ROWS_PER_SUB)])

    return kern(table, gidx, didx, zeros_blk)


def _tc_mlp_bn(parts, node_feats, W1, b1, W2, b2, Wres, bres, gamma, beta):
    def body(parts_r, nf_r, W1_r, b1_r, W2_r, b2_r, Wres_r, bres_r,
             gamma_r, beta_r, out_r):
        agg = parts_r[0, :N_NODES, :] + parts_r[1, :N_NODES, :]
        h1 = jnp.maximum(
            jnp.dot(agg, W1_r[...], preferred_element_type=jnp.float32)
            + b1_r[...], 0.0)
        h = (jnp.dot(h1, W2_r[...], preferred_element_type=jnp.float32)
             + b2_r[...])
        res = (jnp.dot(nf_r[...], Wres_r[...],
                       preferred_element_type=jnp.float32) + bres_r[...])
        h = h + res
        mean = jnp.mean(h, axis=0, keepdims=True)
        var = jnp.mean((h - mean) ** 2, axis=0, keepdims=True)
        out_r[...] = ((h - mean) * lax.rsqrt(var + 1e-5) * gamma_r[...]
                      + beta_r[...])

    return pl.pallas_call(
        body,
        out_shape=jax.ShapeDtypeStruct((N_NODES, D), jnp.float32),
    )(parts, node_feats, W1, b1, W2, b2, Wres, bres, gamma, beta)


@jax.jit
def kernel(node_feats, edge_index, edge_feat_0, edge_feat_1,
           emb0, emb1, W1, b1, W2, b2, Wres, bres, gamma, beta):
    src = edge_index[0].astype(jnp.int32)
    dst = edge_index[1].astype(jnp.int32)
    eidx = (N_NODES + edge_feat_0.astype(jnp.int32) * 3
            + edge_feat_1.astype(jnp.int32))

    emb_pair = (emb0[:, None, :] + emb1[None, :, :]).reshape(18, D)
    table = jnp.concatenate([node_feats, emb_pair], axis=0)

    per = N_EDGES // NW
    g_pad = jnp.zeros((NW, PAD_W), jnp.int32)
    d_pad = jnp.full((NW, PAD_W), TRASH_ROW, jnp.int32)
    gidx = jnp.concatenate(
        [src.reshape(NW, per), eidx.reshape(NW, per), g_pad],
        axis=1).reshape(NW, CHUNKS, K)
    didx = jnp.concatenate(
        [dst.reshape(NW, per), dst.reshape(NW, per), d_pad],
        axis=1).reshape(NW, CHUNKS, K)

    zeros_blk = jnp.zeros((ROWS_PER_SUB, D), jnp.float32)
    parts = _sc_segment_sum(table, gidx, didx, zeros_blk)

    b1_2 = b1.reshape(1, 2 * D)
    b2_2 = b2.reshape(1, D)
    bres_2 = bres.reshape(1, D)
    gamma_2 = gamma.reshape(1, D)
    beta_2 = beta.reshape(1, D)
    return _tc_mlp_bn(parts, node_feats, W1, b1_2, W2, b2_2,
                      Wres, bres_2, gamma_2, beta_2)
